# Initial kernel scaffold; baseline (speedup 1.0000x reference)
import jax
import jax.numpy as jnp
from jax.experimental import pallas as pl


def kernel(table, W1, b1, W2, b2, gumbel):
    raise NotImplementedError("write your pallas kernel here")


# speedup vs baseline: 17.3497x; 17.3497x over previous
import functools

import jax
import jax.numpy as jnp
from jax import lax
from jax.experimental import pallas as pl
from jax.experimental.pallas import tpu as pltpu
from jax.experimental.pallas import tpu_sc as plsc

_B, _R, _COL = 4, 512, 128
_BLOCKS, _CAP = 64, 16
_L = 16
_NCH = _BLOCKS // _L


def _mlp_body(x_ref, w1_ref, b1_ref, w2_ref, b2_ref, g_ref, z_ref):
    h = lax.dot_general(
        x_ref[...], w1_ref[...], (((1,), (1,)), ((), ())),
        preferred_element_type=jnp.float32)
    h = jnp.maximum(h + b1_ref[...], 0.0)
    z = lax.dot_general(
        h, w2_ref[...], (((1,), (1,)), ((), ())),
        preferred_element_type=jnp.float32)
    z = jnp.maximum(z + b2_ref[...], 0.0)
    z_ref[...] = z + g_ref[...]


def _scores(x, w1, b1, w2, b2, g):
    return pl.pallas_call(
        _mlp_body,
        out_shape=jax.ShapeDtypeStruct((_B * _R, _BLOCKS), jnp.float32),
    )(x, w1, b1, w2, b2, g)


def _shuffle(a, perm):
    dn = lax.GatherDimensionNumbers(
        offset_dims=(), collapsed_slice_dims=(0,), start_index_map=(0,))
    return lax.gather(a, perm[:, None], dn, (1,),
                      mode=lax.GatherScatterMode.PROMISE_IN_BOUNDS)


def _route_body(z_hbm, out_hbm, z_v, out_v):
    wid = lax.axis_index("s") * 2 + lax.axis_index("c")

    @pl.when(wid < _B)
    def _():
        pltpu.sync_copy(z_hbm.at[wid], z_v)
        iota = lax.iota(jnp.int32, _L)
        perms = [iota ^ sh for sh in (8, 4, 2, 1)]
        neg = jnp.full((_L,), -1e30, jnp.float32)
        big = jnp.full((_L,), 1 << 20, jnp.int32)
        one_i = jnp.full((_L,), 1, jnp.int32)
        zero_i = jnp.full((_L,), 0, jnp.int32)

        def step(t, counts):
            base = t * _BLOCKS
            vs = [z_v[pl.ds(base + _L * j, _L)] for j in range(_NCH)]
            ms = [jnp.where(counts[j] < _CAP, vs[j], neg) for j in range(_NCH)]
            mx = jnp.maximum(jnp.maximum(ms[0], ms[1]),
                             jnp.maximum(ms[2], ms[3]))
            for p in perms:
                mx = jnp.maximum(mx, _shuffle(mx, p))
            ids = [jnp.where(ms[j] == mx, iota + _L * j, big)
                   for j in range(_NCH)]
            mn = jnp.minimum(jnp.minimum(ids[0], ids[1]),
                             jnp.minimum(ids[2], ids[3]))
            for p in perms:
                mn = jnp.minimum(mn, _shuffle(mn, p))
            new_counts = []
            for j in range(_NCH):
                hit = (iota + _L * j) == mn
                out_v[pl.ds(base + _L * j, _L)] = jnp.where(hit, 1.0, 0.0)
                new_counts.append(counts[j] + jnp.where(hit, one_i, zero_i))
            return tuple(new_counts)

        zero = jnp.zeros((_L,), jnp.int32)
        lax.fori_loop(0, _R, step, (zero, zero, zero, zero))
        pltpu.sync_copy(out_v, out_hbm.at[wid])


def _route(z):
    route = functools.partial(
        pl.kernel,
        mesh=plsc.VectorSubcoreMesh(core_axis_name="c", subcore_axis_name="s"),
        out_type=jax.ShapeDtypeStruct((_B, _R * _BLOCKS), jnp.float32),
        scratch_types=[
            pltpu.VMEM((_R * _BLOCKS,), jnp.float32),
            pltpu.VMEM((_R * _BLOCKS,), jnp.float32),
        ],
    )(_route_body)
    return route(z)


def kernel(table, W1, b1, W2, b2, gumbel):
    x = table.reshape(_B * _R, _COL)
    g = gumbel.reshape(_B * _R, _BLOCKS)
    z = _scores(x, W1, b1.reshape(1, 32), W2, b2.reshape(1, _BLOCKS), g)
    out = _route(z.reshape(_B, _R * _BLOCKS))
    return out.reshape(_B, _R, _BLOCKS)


# speedup vs baseline: 17.5332x; 1.0106x over previous
import functools

import jax
import jax.numpy as jnp
from jax import lax
from jax.experimental import pallas as pl
from jax.experimental.pallas import tpu as pltpu
from jax.experimental.pallas import tpu_sc as plsc

_B, _R, _COL = 4, 512, 128
_BLOCKS, _CAP = 64, 16
_L = 16
_NCH = _BLOCKS // _L


def _mlp_body(x_ref, w1_ref, b1_ref, w2_ref, b2_ref, g_ref, z_ref):
    x = x_ref[...].reshape(_B * _R, _COL)
    h = lax.dot_general(
        x, w1_ref[...], (((1,), (1,)), ((), ())),
        preferred_element_type=jnp.float32)
    h = jnp.maximum(h + b1_ref[...], 0.0)
    z = lax.dot_general(
        h, w2_ref[...], (((1,), (1,)), ((), ())),
        preferred_element_type=jnp.float32)
    z = jnp.maximum(z + b2_ref[...], 0.0)
    z_ref[:, 0:_BLOCKS] = z + g_ref[...].reshape(_B * _R, _BLOCKS)


def _scores(table, w1, b1, w2, b2, g):
    return pl.pallas_call(
        _mlp_body,
        out_shape=jax.ShapeDtypeStruct((_B * _R, 128), jnp.float32),
    )(table, w1, b1, w2, b2, g)


def _shuffle(a, perm):
    dn = lax.GatherDimensionNumbers(
        offset_dims=(), collapsed_slice_dims=(0,), start_index_map=(0,))
    return lax.gather(a, perm[:, None], dn, (1,),
                      mode=lax.GatherScatterMode.PROMISE_IN_BOUNDS)


def _route_body(z_hbm, out_hbm, z_v, out_v):
    wid = lax.axis_index("s") * 2 + lax.axis_index("c")

    @pl.when(wid < _B)
    def _():
        pltpu.sync_copy(z_hbm.at[pl.ds(wid * _R, _R)], z_v)
        iota = lax.iota(jnp.int32, _L)
        perms = [iota ^ sh for sh in (8, 4, 2, 1)]
        neg = jnp.full((_L,), -1e30, jnp.float32)
        big = jnp.full((_L,), 1 << 20, jnp.int32)
        one_i = jnp.full((_L,), 1, jnp.int32)
        zero_i = jnp.full((_L,), 0, jnp.int32)

        def step(t, counts):
            vs = [z_v[t, pl.ds(_L * j, _L)] for j in range(_NCH)]
            ms = [jnp.where(counts[j] < _CAP, vs[j], neg) for j in range(_NCH)]
            mx = jnp.maximum(jnp.maximum(ms[0], ms[1]),
                             jnp.maximum(ms[2], ms[3]))
            for p in perms:
                mx = jnp.maximum(mx, _shuffle(mx, p))
            ids = [jnp.where(ms[j] == mx, iota + _L * j, big)
                   for j in range(_NCH)]
            mn = jnp.minimum(jnp.minimum(ids[0], ids[1]),
                             jnp.minimum(ids[2], ids[3]))
            for p in perms:
                mn = jnp.minimum(mn, _shuffle(mn, p))
            new_counts = []
            for j in range(_NCH):
                hit = (iota + _L * j) == mn
                out_v[t, pl.ds(_L * j, _L)] = jnp.where(hit, 1.0, 0.0)
                new_counts.append(counts[j] + jnp.where(hit, one_i, zero_i))
            return tuple(new_counts)

        zero = jnp.zeros((_L,), jnp.int32)
        lax.fori_loop(0, _R, step, (zero, zero, zero, zero))
        pltpu.sync_copy(out_v, out_hbm.at[wid])


def _route(z):
    route = functools.partial(
        pl.kernel,
        mesh=plsc.VectorSubcoreMesh(core_axis_name="c", subcore_axis_name="s"),
        out_type=jax.ShapeDtypeStruct((_B, _R, _BLOCKS), jnp.float32),
        scratch_types=[
            pltpu.VMEM((_R, 128), jnp.float32),
            pltpu.VMEM((_R, _BLOCKS), jnp.float32),
        ],
    )(_route_body)
    return route(z)


def kernel(table, W1, b1, W2, b2, gumbel):
    z = _scores(table, W1, b1.reshape(1, 32), W2, b2.reshape(1, _BLOCKS),
                gumbel)
    return _route(z)


# speedup vs baseline: 17.5542x; 1.0012x over previous
import functools

import jax
import jax.numpy as jnp
from jax import lax
from jax.experimental import pallas as pl
from jax.experimental.pallas import tpu as pltpu
from jax.experimental.pallas import tpu_sc as plsc

_B, _R, _COL = 4, 512, 128
_BLOCKS, _CAP = 64, 16
_L = 16
_NCH = _BLOCKS // _L


def _mlp_body(x_ref, w1_ref, b1_ref, w2_ref, b2_ref, g_ref, z_ref):
    x = x_ref[...].reshape(_B * _R, _COL)
    h = lax.dot_general(
        x, w1_ref[...], (((1,), (1,)), ((), ())),
        preferred_element_type=jnp.float32)
    h = jnp.maximum(h + b1_ref[...], 0.0)
    z = lax.dot_general(
        h, w2_ref[...], (((1,), (1,)), ((), ())),
        preferred_element_type=jnp.float32)
    z = jnp.maximum(z + b2_ref[...], 0.0)
    z_ref[:, 0:_BLOCKS] = z + g_ref[...].reshape(_B * _R, _BLOCKS)


def _scores(table, w1, b1, w2, b2, g):
    return pl.pallas_call(
        _mlp_body,
        out_shape=jax.ShapeDtypeStruct((_B * _R, 128), jnp.float32),
    )(table, w1, b1, w2, b2, g)


def _shuffle(a, perm):
    dn = lax.GatherDimensionNumbers(
        offset_dims=(), collapsed_slice_dims=(0,), start_index_map=(0,))
    return lax.gather(a, perm[:, None], dn, (1,),
                      mode=lax.GatherScatterMode.PROMISE_IN_BOUNDS)


def _route_body(z_hbm, out_hbm, z_v, out_v):
    wid = lax.axis_index("s") * 2 + lax.axis_index("c")

    @pl.when(wid < _B)
    def _():
        pltpu.sync_copy(z_hbm.at[pl.ds(wid * _R, _R)], z_v)
        iota = lax.iota(jnp.int32, _L)
        perms = [iota ^ sh for sh in (8, 4, 2, 1)]
        neg = jnp.full((_L,), -1e30, jnp.float32)
        big = jnp.full((_L,), 1 << 20, jnp.int32)
        cap = jnp.full((_L,), float(_CAP), jnp.float32)

        def step(t, counts):
            vs = [z_v[t, pl.ds(_L * j, _L)] for j in range(_NCH)]
            ms = [jnp.where(counts[j] < cap, vs[j], neg) for j in range(_NCH)]
            mx = jnp.maximum(jnp.maximum(ms[0], ms[1]),
                             jnp.maximum(ms[2], ms[3]))
            for p in perms:
                mx = jnp.maximum(mx, _shuffle(mx, p))
            ids = [jnp.where(ms[j] == mx, iota + _L * j, big)
                   for j in range(_NCH)]
            mn = jnp.minimum(jnp.minimum(ids[0], ids[1]),
                             jnp.minimum(ids[2], ids[3]))
            for p in perms:
                mn = jnp.minimum(mn, _shuffle(mn, p))
            new_counts = []
            for j in range(_NCH):
                oh = jnp.where((iota + _L * j) == mn, 1.0, 0.0)
                out_v[t, pl.ds(_L * j, _L)] = oh
                new_counts.append(counts[j] + oh)
            return tuple(new_counts)

        zero = jnp.zeros((_L,), jnp.float32)
        plsc.parallel_loop(0, _R, unroll=4,
                           carry=(zero, zero, zero, zero))(step)
        pltpu.sync_copy(out_v, out_hbm.at[wid])


def _route(z):
    route = functools.partial(
        pl.kernel,
        mesh=plsc.VectorSubcoreMesh(core_axis_name="c", subcore_axis_name="s"),
        out_type=jax.ShapeDtypeStruct((_B, _R, _BLOCKS), jnp.float32),
        scratch_types=[
            pltpu.VMEM((_R, 128), jnp.float32),
            pltpu.VMEM((_R, _BLOCKS), jnp.float32),
        ],
    )(_route_body)
    return route(z)


def kernel(table, W1, b1, W2, b2, gumbel):
    z = _scores(table, W1, b1.reshape(1, 32), W2, b2.reshape(1, _BLOCKS),
                gumbel)
    return _route(z)
